# Initial kernel scaffold; baseline (speedup 1.0000x reference)
import jax
import jax.numpy as jnp
from jax.experimental import pallas as pl


def kernel(feat0, feat1, feat2, conv_w, conv_b, cls_w, cls_b, bbox_w, bbox_b):
    raise NotImplementedError("write your pallas kernel here")


# speedup vs baseline: 14.3922x; 14.3922x over previous
import functools

import numpy as np
import jax
import jax.numpy as jnp
from jax import lax
from jax.experimental import pallas as pl
from jax.experimental.pallas import tpu as pltpu

_IMG = 512.0
_STRIDES = (8, 16, 32)
_SIZES = (64.0, 128.0, 256.0)
_RATIOS = np.array([0.5, 1.0, 2.0], dtype=np.float32)
_PRE = 1000
_POST = 300
_TH = 0.7
_A = 3
_C = 256
_CLIP = float(np.log(1000.0 / 16.0))

_NEG = float("-inf")

_INTERPRET = False


def _conv_head_body(x_ref, w9_ref, b_ref, hw_ref, hb_ref, wh_ref,
                    s_ref, box_ref, *, H, W, stride, bh):
    HW = bh * W
    row0 = pl.program_id(1) * bh
    x = x_ref[0, pl.dslice(pl.multiple_of(row0, bh), bh + 2), :, :]
    slots = []
    for k in range(9):
        ky, kx = divmod(k, 3)
        xs = x[ky:ky + bh, kx:kx + W, :].reshape(HW, _C)
        cur = jnp.dot(xs, w9_ref[k], preferred_element_type=jnp.float32)
        i = 0
        while i < len(slots) and slots[i] is not None:
            cur = slots[i] + cur
            slots[i] = None
            i += 1
        if i == len(slots):
            slots.append(cur)
        else:
            slots[i] = cur
    acc = None
    for s in reversed(slots):
        if s is not None:
            acc = s if acc is None else acc + s
    t = jnp.maximum(acc + b_ref[...], 0.0)
    heads = jnp.dot(t, hw_ref[...], preferred_element_type=jnp.float32) \
        + hb_ref[...]
    logits = heads[:, 0:3]
    dx = heads[:, 3:6]
    dy = heads[:, 6:9]
    dw = heads[:, 9:12]
    dh = heads[:, 12:15]
    scores = 1.0 / (1.0 + jnp.exp(-logits))

    p = lax.broadcasted_iota(jnp.int32, (HW, 1), 0)
    px = (p % W).astype(jnp.float32)
    py = (p // W + row0).astype(jnp.float32)
    cxa = (px + 0.5) * stride
    cya = (py + 0.5) * stride
    wa = wh_ref[0:1, :]
    ha = wh_ref[1:2, :]
    cx = dx * wa + cxa
    cy = dy * ha + cya
    w = jnp.exp(jnp.minimum(dw, _CLIP)) * wa
    h = jnp.exp(jnp.minimum(dh, _CLIP)) * ha
    x1 = jnp.clip(cx - 0.5 * w, 0.0, _IMG)
    y1 = jnp.clip(cy - 0.5 * h, 0.0, _IMG)
    x2 = jnp.clip(cx + 0.5 * w, 0.0, _IMG)
    y2 = jnp.clip(cy + 0.5 * h, 0.0, _IMG)

    s_ref[0] = scores.T
    box_ref[0] = jnp.concatenate(
        [x1.T, y1.T, x2.T, y2.T], axis=0)


def _conv_head(xpad, w9, b2, head_w, head_b, wh, H, W, stride):
    B = xpad.shape[0]
    HW = H * W
    nblk = 4 if H >= 64 else 1
    bh = H // nblk
    body = functools.partial(_conv_head_body, H=H, W=W, stride=stride, bh=bh)
    return pl.pallas_call(
        body,
        grid=(B, nblk),
        in_specs=[
            pl.BlockSpec((1, H + 2, W + 2, _C), lambda b, i: (b, 0, 0, 0)),
            pl.BlockSpec((9, _C, _C), lambda b, i: (0, 0, 0)),
            pl.BlockSpec((1, _C), lambda b, i: (0, 0)),
            pl.BlockSpec((_C, 15), lambda b, i: (0, 0)),
            pl.BlockSpec((1, 15), lambda b, i: (0, 0)),
            pl.BlockSpec((2, 3), lambda b, i: (0, 0)),
        ],
        out_specs=[
            pl.BlockSpec((1, 3, bh * W), lambda b, i: (b, 0, i)),
            pl.BlockSpec((1, 12, bh * W), lambda b, i: (b, 0, i)),
        ],
        out_shape=[
            jax.ShapeDtypeStruct((B, 3, HW), jnp.float32),
            jax.ShapeDtypeStruct((B, 12, HW), jnp.float32),
        ],
        interpret=_INTERPRET,
    )(xpad, w9, b2, head_w, head_b, wh)


def _sort_body(k_ref, ks_ref, vs_ref, *, N):
    R = N // 128
    key = k_ref[0]
    r = lax.broadcasted_iota(jnp.int32, (R, 128), 0)
    c = lax.broadcasted_iota(jnp.int32, (R, 128), 1)
    idx = r * 128 + c
    val = idx

    k = 2
    while k <= N:
        j = k // 2
        while j >= 1:
            if j >= 128:
                m = j // 128
                lower = (r & m) == 0
                pk = jnp.where(lower, jnp.roll(key, -m, axis=0),
                               jnp.roll(key, m, axis=0))
                pv = jnp.where(lower, jnp.roll(val, -m, axis=0),
                               jnp.roll(val, m, axis=0))
            else:
                lower = (c & j) == 0
                pk = jnp.where(lower, jnp.roll(key, -j, axis=1),
                               jnp.roll(key, j, axis=1))
                pv = jnp.where(lower, jnp.roll(val, -j, axis=1),
                               jnp.roll(val, j, axis=1))
            lo_k = jnp.where(lower, key, pk)
            hi_k = jnp.where(lower, pk, key)
            lo_v = jnp.where(lower, val, pv)
            hi_v = jnp.where(lower, pv, val)
            good = (lo_k > hi_k) | ((lo_k == hi_k) & (lo_v < hi_v))
            want_desc = (idx & k) == 0
            take_own = good == want_desc
            key = jnp.where(take_own, key, pk)
            val = jnp.where(take_own, val, pv)
            j //= 2
        k *= 2

    ks_ref[0] = key
    vs_ref[0] = val


def _sort_desc(scores_pad):
    B, R, _ = scores_pad.shape
    N = R * 128
    body = functools.partial(_sort_body, N=N)
    return pl.pallas_call(
        body,
        grid=(B,),
        in_specs=[pl.BlockSpec((1, R, 128), lambda b: (b, 0, 0))],
        out_specs=[pl.BlockSpec((1, R, 128), lambda b: (b, 0, 0)),
                   pl.BlockSpec((1, R, 128), lambda b: (b, 0, 0))],
        out_shape=[jax.ShapeDtypeStruct((B, R, 128), jnp.float32),
                   jax.ShapeDtypeStruct((B, R, 128), jnp.int32)],
        interpret=_INTERPRET,
    )(scores_pad)


_NMS_N = 2816
_NMS_NB = _NMS_N // 128
_NMS_VALID = 2768


def _cumsum_lanes(x):
    n = x.shape[-1]
    s = 1
    while s < n:
        shifted = jnp.concatenate(
            [jnp.zeros((1, s), x.dtype), x[:, :n - s]], axis=1)
        x = x + shifted
        s *= 2
    return x


def _nms_body(brow_ref, bcol_ref, s_ref, out_ref, keep_ref, av_ref):
    x1a = bcol_ref[0, 0:1, :]
    y1a = bcol_ref[0, 1:2, :]
    x2a = bcol_ref[0, 2:3, :]
    y2a = bcol_ref[0, 3:4, :]
    s = s_ref[0]
    area_a = (x2a - x1a) * (y2a - y1a)

    lane = lax.broadcasted_iota(jnp.int32, (1, 128), 1)
    ri = lax.broadcasted_iota(jnp.int32, (128, 128), 0)
    ci = lax.broadcasted_iota(jnp.int32, (128, 128), 1)
    ident = jnp.where(ri == ci, 1.0, 0.0)

    keep_ref[...] = jnp.zeros((1, _NMS_N), jnp.float32)

    def block_step(blk, carry):
        base = pl.multiple_of(blk * 128, 128)
        keepm = keep_ref[...]
        bx1 = brow_ref[0, pl.dslice(base, 128), 0:1]
        by1 = brow_ref[0, pl.dslice(base, 128), 1:2]
        bx2 = brow_ref[0, pl.dslice(base, 128), 2:3]
        by2 = brow_ref[0, pl.dslice(base, 128), 3:4]
        area_b = (bx2 - bx1) * (by2 - by1)
        iw = jnp.minimum(bx2, x2a) - jnp.maximum(bx1, x1a)
        ih = jnp.minimum(by2, y2a) - jnp.maximum(by1, y1a)
        inter = jnp.maximum(iw, 0.0) * jnp.maximum(ih, 0.0)
        iou = inter / (area_b + area_a - inter + 1e-9)
        overf = jnp.where(iou > _TH, 1.0, 0.0) * keepm
        sup_col = jnp.max(overf, axis=1, keepdims=True)
        sup_row = lax.dot_general(sup_col, ident, (((0,), (0,)), ((), ())),
                                  preferred_element_type=jnp.float32)
        tx1 = bcol_ref[0, 0:1, pl.dslice(base, 128)]
        ty1 = bcol_ref[0, 1:2, pl.dslice(base, 128)]
        tx2 = bcol_ref[0, 2:3, pl.dslice(base, 128)]
        ty2 = bcol_ref[0, 3:4, pl.dslice(base, 128)]
        area_t = (tx2 - tx1) * (ty2 - ty1)
        iw2 = jnp.minimum(bx2, tx2) - jnp.maximum(bx1, tx1)
        ih2 = jnp.minimum(by2, ty2) - jnp.maximum(by1, ty1)
        inter2 = jnp.maximum(iw2, 0.0) * jnp.maximum(ih2, 0.0)
        iou2 = inter2 / (area_b + area_t - inter2 + 1e-9)
        overb = jnp.where((iou2 > _TH) & (ci > ri), 1.0, 0.0)

        gidx = base + lane
        av = jnp.where((gidx < _NMS_VALID) & (sup_row < 0.5), 1.0, 0.0)
        av_ref[...] = av
        for i in range(128):
            a_i = av_ref[0, i]
            row = overb[i:i + 1, :]
            av = av * (1.0 - row * a_i)
            av_ref[...] = av
        keep_ref[0:1, pl.dslice(base, 128)] = av
        return carry

    lax.fori_loop(0, _NMS_NB, block_step, 0, unroll=False)

    keepf = keep_ref[...]
    keep = keepf > 0.5
    validf = (lax.broadcasted_iota(jnp.int32, (1, _NMS_N), 1)
              < _NMS_VALID).astype(jnp.float32)
    supf = validf * (1.0 - keepf)
    kc = _cumsum_lanes(keepf)
    krank = kc - keepf
    kk = kc[:, _NMS_N - 1:_NMS_N]
    scn = _cumsum_lanes(supf)
    srank = scn - supf
    pos = keepf * krank + supf * (kk + srank) + (1.0 - keepf - supf) * 1e9

    rows = lax.broadcasted_iota(jnp.int32, (_POST, 1), 0).astype(jnp.float32)
    oh = (pos == rows).astype(jnp.float32)
    cols = []
    s_kept = jnp.where(keep, s, 0.0)
    for cdata in (x1a, y1a, x2a, y2a, s_kept):
        cols.append(jnp.sum(oh * cdata, axis=1, keepdims=True))
    out = jnp.concatenate(cols, axis=1)
    srow = jnp.where(rows >= kk, _NEG, out[:, 4:5])
    out = jnp.concatenate([out[:, 0:4], srow], axis=1)
    out_ref[0] = out


def _nms_topk(brow, bcol, s):
    B = brow.shape[0]
    return pl.pallas_call(
        _nms_body,
        grid=(B,),
        in_specs=[
            pl.BlockSpec((1, _NMS_N, 4), lambda b: (b, 0, 0)),
            pl.BlockSpec((1, 4, _NMS_N), lambda b: (b, 0, 0)),
            pl.BlockSpec((1, 1, _NMS_N), lambda b: (b, 0, 0)),
        ],
        out_specs=pl.BlockSpec((1, _POST, 5), lambda b: (b, 0, 0)),
        out_shape=jax.ShapeDtypeStruct((B, _POST, 5), jnp.float32),
        scratch_shapes=[pltpu.VMEM((1, _NMS_N), jnp.float32),
                        pltpu.VMEM((1, 128), jnp.float32)],
        interpret=_INTERPRET,
    )(brow, bcol, s)


def _level_params(conv_w, conv_b, cls_w, cls_b, bbox_w, bbox_b):
    w9 = jnp.transpose(conv_w, (2, 3, 1, 0)).reshape(9, _C, _C)
    b2 = conv_b.reshape(1, _C)
    cw = cls_w.reshape(_A, _C).T
    bw = bbox_w.reshape(_A * 4, _C).T
    parts = [cw] + [bw[:, c::4] for c in range(4)]
    head_w = jnp.concatenate(parts, axis=1)
    cb = cls_b.reshape(1, _A)
    bb = bbox_b.reshape(1, _A * 4)
    head_b = jnp.concatenate([cb] + [bb[:, c::4] for c in range(4)], axis=1)
    return w9, b2, head_w, head_b


def kernel(feat0, feat1, feat2, conv_w, conv_b, cls_w, cls_b, bbox_w, bbox_b):
    w9, b2, head_w, head_b = _level_params(
        conv_w, conv_b, cls_w, cls_b, bbox_w, bbox_b)
    feats = (feat0, feat1, feat2)
    B = feat0.shape[0]

    wa_np = (np.array(_SIZES, np.float32)[:, None]
             / np.sqrt(_RATIOS)[None, :])
    ha_np = (np.array(_SIZES, np.float32)[:, None]
             * np.sqrt(_RATIOS)[None, :])

    sc_sorted = []
    idx_sorted = []
    box_tables = []
    offset = 0
    for lvl, f in enumerate(feats):
        H, W = f.shape[2], f.shape[3]
        HW = H * W
        x = jnp.transpose(f, (0, 2, 3, 1))
        xpad = jnp.pad(x, ((0, 0), (1, 1), (1, 1), (0, 0)))
        wh = jnp.asarray(
            np.stack([wa_np[lvl], ha_np[lvl]], axis=0))
        scores, boxes = _conv_head(xpad, w9, b2, head_w, head_b, wh,
                                   H, W, _STRIDES[lvl])
        sflat = jnp.transpose(scores, (0, 2, 1)).reshape(B, HW * _A)
        bflat = jnp.transpose(boxes.reshape(B, 4, _A, HW),
                              (0, 3, 2, 1)).reshape(B, HW * _A, 4)
        box_tables.append(bflat)

        n = HW * _A
        npad = 1 << int(np.ceil(np.log2(n)))
        spad = jnp.pad(sflat, ((0, 0), (0, npad - n)),
                       constant_values=_NEG).reshape(B, npad // 128, 128)
        ks, vs = _sort_desc(spad)
        k = min(_PRE, n)
        sc_sorted.append(ks.reshape(B, npad)[:, :k])
        idx_sorted.append(vs.reshape(B, npad)[:, :k] + offset)
        offset += n

    s_cat = jnp.concatenate(sc_sorted, axis=1)
    i_cat = jnp.concatenate(idx_sorted, axis=1)
    table = jnp.concatenate(box_tables, axis=1)

    boxes_c = jnp.take_along_axis(table, i_cat[:, :, None], axis=1)

    ncand = s_cat.shape[1]
    npad = 4096
    spad = jnp.pad(s_cat, ((0, 0), (0, npad - ncand)),
                   constant_values=_NEG).reshape(B, npad // 128, 128)
    ks, vs = _sort_desc(spad)
    s_sorted = ks.reshape(B, npad)[:, :ncand]
    perm = vs.reshape(B, npad)[:, :ncand]
    boxes_s = jnp.take_along_axis(boxes_c, perm[:, :, None], axis=1)

    brow = jnp.pad(boxes_s, ((0, 0), (0, _NMS_N - ncand), (0, 0)))
    s_in = jnp.pad(s_sorted, ((0, 0), (0, _NMS_N - ncand)),
                   constant_values=_NEG).reshape(B, 1, _NMS_N)
    bcol = jnp.transpose(brow, (0, 2, 1))
    return _nms_topk(brow, bcol, s_in)


# speedup vs baseline: 14.4205x; 1.0020x over previous
import functools

import numpy as np
import jax
import jax.numpy as jnp
from jax import lax
from jax.experimental import pallas as pl
from jax.experimental.pallas import tpu as pltpu
from jax.experimental.pallas import tpu_sc as plsc

_IMG = 512.0
_STRIDES = (8, 16, 32)
_SIZES = (64.0, 128.0, 256.0)
_RATIOS = np.array([0.5, 1.0, 2.0], dtype=np.float32)
_PRE = 1000
_POST = 300
_TH = 0.7
_A = 3
_C = 256
_CLIP = float(np.log(1000.0 / 16.0))

_NEG = float("-inf")

_INTERPRET = False


def _conv_head_body(x_ref, w9_ref, b_ref, hw_ref, hb_ref, wh_ref,
                    s_ref, box_ref, *, H, W, stride, bh):
    HW = bh * W
    row0 = pl.program_id(1) * bh
    x = x_ref[0, pl.dslice(pl.multiple_of(row0, bh), bh + 2), :, :]
    slots = []
    for k in range(9):
        ky, kx = divmod(k, 3)
        xs = x[ky:ky + bh, kx:kx + W, :].reshape(HW, _C)
        cur = jnp.dot(xs, w9_ref[k], preferred_element_type=jnp.float32)
        i = 0
        while i < len(slots) and slots[i] is not None:
            cur = slots[i] + cur
            slots[i] = None
            i += 1
        if i == len(slots):
            slots.append(cur)
        else:
            slots[i] = cur
    acc = None
    for s in reversed(slots):
        if s is not None:
            acc = s if acc is None else acc + s
    t = jnp.maximum(acc + b_ref[...], 0.0)
    heads = jnp.dot(t, hw_ref[...], preferred_element_type=jnp.float32) \
        + hb_ref[...]
    logits = heads[:, 0:3]
    dx = heads[:, 3:6]
    dy = heads[:, 6:9]
    dw = heads[:, 9:12]
    dh = heads[:, 12:15]
    scores = 1.0 / (1.0 + jnp.exp(-logits))

    p = lax.broadcasted_iota(jnp.int32, (HW, 1), 0)
    px = (p % W).astype(jnp.float32)
    py = (p // W + row0).astype(jnp.float32)
    cxa = (px + 0.5) * stride
    cya = (py + 0.5) * stride
    wa = wh_ref[0:1, :]
    ha = wh_ref[1:2, :]
    cx = dx * wa + cxa
    cy = dy * ha + cya
    w = jnp.exp(jnp.minimum(dw, _CLIP)) * wa
    h = jnp.exp(jnp.minimum(dh, _CLIP)) * ha
    x1 = jnp.clip(cx - 0.5 * w, 0.0, _IMG)
    y1 = jnp.clip(cy - 0.5 * h, 0.0, _IMG)
    x2 = jnp.clip(cx + 0.5 * w, 0.0, _IMG)
    y2 = jnp.clip(cy + 0.5 * h, 0.0, _IMG)

    s_ref[0] = scores.T
    box_ref[0] = jnp.concatenate(
        [x1.T, y1.T, x2.T, y2.T], axis=0)


def _conv_head(xpad, w9, b2, head_w, head_b, wh, H, W, stride):
    B = xpad.shape[0]
    HW = H * W
    nblk = 4 if H >= 64 else 1
    bh = H // nblk
    body = functools.partial(_conv_head_body, H=H, W=W, stride=stride, bh=bh)
    return pl.pallas_call(
        body,
        grid=(B, nblk),
        in_specs=[
            pl.BlockSpec((1, H + 2, W + 2, _C), lambda b, i: (b, 0, 0, 0)),
            pl.BlockSpec((9, _C, _C), lambda b, i: (0, 0, 0)),
            pl.BlockSpec((1, _C), lambda b, i: (0, 0)),
            pl.BlockSpec((_C, 15), lambda b, i: (0, 0)),
            pl.BlockSpec((1, 15), lambda b, i: (0, 0)),
            pl.BlockSpec((2, 3), lambda b, i: (0, 0)),
        ],
        out_specs=[
            pl.BlockSpec((1, 3, bh * W), lambda b, i: (b, 0, i)),
            pl.BlockSpec((1, 12, bh * W), lambda b, i: (b, 0, i)),
        ],
        out_shape=[
            jax.ShapeDtypeStruct((B, 3, HW), jnp.float32),
            jax.ShapeDtypeStruct((B, 12, HW), jnp.float32),
        ],
        interpret=_INTERPRET,
    )(xpad, w9, b2, head_w, head_b, wh)


def _sort_body(k_ref, ks_ref, vs_ref, *, N):
    R = N // 128
    key = k_ref[0]
    r = lax.broadcasted_iota(jnp.int32, (R, 128), 0)
    c = lax.broadcasted_iota(jnp.int32, (R, 128), 1)
    idx = r * 128 + c
    val = idx

    k = 2
    while k <= N:
        j = k // 2
        while j >= 1:
            if j >= 128:
                m = j // 128
                lower = (r & m) == 0
                pk = jnp.where(lower, jnp.roll(key, -m, axis=0),
                               jnp.roll(key, m, axis=0))
                pv = jnp.where(lower, jnp.roll(val, -m, axis=0),
                               jnp.roll(val, m, axis=0))
            else:
                lower = (c & j) == 0
                pk = jnp.where(lower, jnp.roll(key, -j, axis=1),
                               jnp.roll(key, j, axis=1))
                pv = jnp.where(lower, jnp.roll(val, -j, axis=1),
                               jnp.roll(val, j, axis=1))
            lo_k = jnp.where(lower, key, pk)
            hi_k = jnp.where(lower, pk, key)
            lo_v = jnp.where(lower, val, pv)
            hi_v = jnp.where(lower, pv, val)
            good = (lo_k > hi_k) | ((lo_k == hi_k) & (lo_v < hi_v))
            want_desc = (idx & k) == 0
            take_own = good == want_desc
            key = jnp.where(take_own, key, pk)
            val = jnp.where(take_own, val, pv)
            j //= 2
        k *= 2

    ks_ref[0] = key
    vs_ref[0] = val


def _sort_desc(scores_pad):
    B, R, _ = scores_pad.shape
    N = R * 128
    body = functools.partial(_sort_body, N=N)
    return pl.pallas_call(
        body,
        grid=(B,),
        in_specs=[pl.BlockSpec((1, R, 128), lambda b: (b, 0, 0))],
        out_specs=[pl.BlockSpec((1, R, 128), lambda b: (b, 0, 0)),
                   pl.BlockSpec((1, R, 128), lambda b: (b, 0, 0))],
        out_shape=[jax.ShapeDtypeStruct((B, R, 128), jnp.float32),
                   jax.ShapeDtypeStruct((B, R, 128), jnp.int32)],
        interpret=_INTERPRET,
    )(scores_pad)


def _sc_gather(table, idx):
    info = plsc.get_sparse_core_info()
    nc, ns = info.num_cores, info.num_subcores
    nw = nc * ns
    btot = idx.shape[0]
    assert btot % (8 * nw) == 0
    bpw = btot // nw
    d = table.shape[1]
    mesh = plsc.VectorSubcoreMesh(core_axis_name="c", subcore_axis_name="s")

    @functools.partial(
        pl.kernel, mesh=mesh,
        out_type=jax.ShapeDtypeStruct((btot, d), jnp.float32),
        scratch_types=[
            pltpu.VMEM((bpw,), jnp.int32),
            pltpu.VMEM((bpw, d), jnp.float32),
            pltpu.SemaphoreType.DMA,
        ],
    )
    def k(table_hbm, idx_hbm, out_hbm, idx_v, rows_v, sem):
        wid = lax.axis_index("s") * nc + lax.axis_index("c")
        base = wid * bpw
        pltpu.sync_copy(idx_hbm.at[pl.ds(base, bpw)], idx_v)
        pltpu.async_copy(table_hbm.at[idx_v], rows_v, sem).wait()
        pltpu.sync_copy(rows_v, out_hbm.at[pl.ds(base, bpw)])

    return k(table, idx)


def _gather_boxes(table_b4, idx_b):
    B, V, _ = table_b4.shape
    n = idx_b.shape[1]
    tab = jnp.pad(table_b4, ((0, 0), (0, 0), (0, 124))).reshape(B * V, 128)
    flat = (idx_b + (jnp.arange(B, dtype=jnp.int32) * V)[:, None]).reshape(-1)
    pad = (-flat.shape[0]) % 256
    flat = jnp.pad(flat, (0, pad))
    out = _sc_gather(tab, flat)
    return out[:B * n, :4].reshape(B, n, 4)


_NMS_N = 2816
_NMS_NB = _NMS_N // 128
_NMS_VALID = 2768


def _cumsum_lanes(x):
    n = x.shape[-1]
    s = 1
    while s < n:
        shifted = jnp.concatenate(
            [jnp.zeros((1, s), x.dtype), x[:, :n - s]], axis=1)
        x = x + shifted
        s *= 2
    return x


def _nms_body(brow_ref, bcol_ref, s_ref, out_ref, keep_ref, av_ref):
    x1a = bcol_ref[0, 0:1, :]
    y1a = bcol_ref[0, 1:2, :]
    x2a = bcol_ref[0, 2:3, :]
    y2a = bcol_ref[0, 3:4, :]
    s = s_ref[0]
    area_a = (x2a - x1a) * (y2a - y1a)

    lane = lax.broadcasted_iota(jnp.int32, (1, 128), 1)
    ri = lax.broadcasted_iota(jnp.int32, (128, 128), 0)
    ci = lax.broadcasted_iota(jnp.int32, (128, 128), 1)
    ident = jnp.where(ri == ci, 1.0, 0.0)

    keep_ref[...] = jnp.zeros((1, _NMS_N), jnp.float32)

    def block_step(blk, carry):
        base = pl.multiple_of(blk * 128, 128)
        keepm = keep_ref[...]
        bx1 = brow_ref[0, pl.dslice(base, 128), 0:1]
        by1 = brow_ref[0, pl.dslice(base, 128), 1:2]
        bx2 = brow_ref[0, pl.dslice(base, 128), 2:3]
        by2 = brow_ref[0, pl.dslice(base, 128), 3:4]
        area_b = (bx2 - bx1) * (by2 - by1)
        iw = jnp.minimum(bx2, x2a) - jnp.maximum(bx1, x1a)
        ih = jnp.minimum(by2, y2a) - jnp.maximum(by1, y1a)
        inter = jnp.maximum(iw, 0.0) * jnp.maximum(ih, 0.0)
        iou = inter / (area_b + area_a - inter + 1e-9)
        overf = jnp.where(iou > _TH, 1.0, 0.0) * keepm
        sup_col = jnp.max(overf, axis=1, keepdims=True)
        sup_row = lax.dot_general(sup_col, ident, (((0,), (0,)), ((), ())),
                                  preferred_element_type=jnp.float32)
        tx1 = bcol_ref[0, 0:1, pl.dslice(base, 128)]
        ty1 = bcol_ref[0, 1:2, pl.dslice(base, 128)]
        tx2 = bcol_ref[0, 2:3, pl.dslice(base, 128)]
        ty2 = bcol_ref[0, 3:4, pl.dslice(base, 128)]
        area_t = (tx2 - tx1) * (ty2 - ty1)
        iw2 = jnp.minimum(bx2, tx2) - jnp.maximum(bx1, tx1)
        ih2 = jnp.minimum(by2, ty2) - jnp.maximum(by1, ty1)
        inter2 = jnp.maximum(iw2, 0.0) * jnp.maximum(ih2, 0.0)
        iou2 = inter2 / (area_b + area_t - inter2 + 1e-9)
        overb = jnp.where((iou2 > _TH) & (ci > ri), 1.0, 0.0)

        gidx = base + lane
        av = jnp.where((gidx < _NMS_VALID) & (sup_row < 0.5), 1.0, 0.0)
        av_ref[...] = av
        for i in range(128):
            a_i = av_ref[0, i]
            row = overb[i:i + 1, :]
            av = av * (1.0 - row * a_i)
            av_ref[...] = av
        keep_ref[0:1, pl.dslice(base, 128)] = av
        return carry

    lax.fori_loop(0, _NMS_NB, block_step, 0, unroll=False)

    keepf = keep_ref[...]
    keep = keepf > 0.5
    validf = (lax.broadcasted_iota(jnp.int32, (1, _NMS_N), 1)
              < _NMS_VALID).astype(jnp.float32)
    supf = validf * (1.0 - keepf)
    kc = _cumsum_lanes(keepf)
    krank = kc - keepf
    kk = kc[:, _NMS_N - 1:_NMS_N]
    scn = _cumsum_lanes(supf)
    srank = scn - supf
    pos = keepf * krank + supf * (kk + srank) + (1.0 - keepf - supf) * 1e9

    rows = lax.broadcasted_iota(jnp.int32, (_POST, 1), 0).astype(jnp.float32)
    oh = (pos == rows).astype(jnp.float32)
    cols = []
    s_kept = jnp.where(keep, s, 0.0)
    for cdata in (x1a, y1a, x2a, y2a, s_kept):
        cols.append(jnp.sum(oh * cdata, axis=1, keepdims=True))
    out = jnp.concatenate(cols, axis=1)
    srow = jnp.where(rows >= kk, _NEG, out[:, 4:5])
    out = jnp.concatenate([out[:, 0:4], srow], axis=1)
    out_ref[0] = out


def _nms_topk(brow, bcol, s):
    B = brow.shape[0]
    return pl.pallas_call(
        _nms_body,
        grid=(B,),
        in_specs=[
            pl.BlockSpec((1, _NMS_N, 4), lambda b: (b, 0, 0)),
            pl.BlockSpec((1, 4, _NMS_N), lambda b: (b, 0, 0)),
            pl.BlockSpec((1, 1, _NMS_N), lambda b: (b, 0, 0)),
        ],
        out_specs=pl.BlockSpec((1, _POST, 5), lambda b: (b, 0, 0)),
        out_shape=jax.ShapeDtypeStruct((B, _POST, 5), jnp.float32),
        scratch_shapes=[pltpu.VMEM((1, _NMS_N), jnp.float32),
                        pltpu.VMEM((1, 128), jnp.float32)],
        interpret=_INTERPRET,
    )(brow, bcol, s)


def _level_params(conv_w, conv_b, cls_w, cls_b, bbox_w, bbox_b):
    w9 = jnp.transpose(conv_w, (2, 3, 1, 0)).reshape(9, _C, _C)
    b2 = conv_b.reshape(1, _C)
    cw = cls_w.reshape(_A, _C).T
    bw = bbox_w.reshape(_A * 4, _C).T
    parts = [cw] + [bw[:, c::4] for c in range(4)]
    head_w = jnp.concatenate(parts, axis=1)
    cb = cls_b.reshape(1, _A)
    bb = bbox_b.reshape(1, _A * 4)
    head_b = jnp.concatenate([cb] + [bb[:, c::4] for c in range(4)], axis=1)
    return w9, b2, head_w, head_b


def kernel(feat0, feat1, feat2, conv_w, conv_b, cls_w, cls_b, bbox_w, bbox_b):
    w9, b2, head_w, head_b = _level_params(
        conv_w, conv_b, cls_w, cls_b, bbox_w, bbox_b)
    feats = (feat0, feat1, feat2)
    B = feat0.shape[0]

    wa_np = (np.array(_SIZES, np.float32)[:, None]
             / np.sqrt(_RATIOS)[None, :])
    ha_np = (np.array(_SIZES, np.float32)[:, None]
             * np.sqrt(_RATIOS)[None, :])

    sc_sorted = []
    idx_sorted = []
    box_tables = []
    offset = 0
    for lvl, f in enumerate(feats):
        H, W = f.shape[2], f.shape[3]
        HW = H * W
        x = jnp.transpose(f, (0, 2, 3, 1))
        xpad = jnp.pad(x, ((0, 0), (1, 1), (1, 1), (0, 0)))
        wh = jnp.asarray(
            np.stack([wa_np[lvl], ha_np[lvl]], axis=0))
        scores, boxes = _conv_head(xpad, w9, b2, head_w, head_b, wh,
                                   H, W, _STRIDES[lvl])
        sflat = jnp.transpose(scores, (0, 2, 1)).reshape(B, HW * _A)
        bflat = jnp.transpose(boxes.reshape(B, 4, _A, HW),
                              (0, 3, 2, 1)).reshape(B, HW * _A, 4)
        box_tables.append(bflat)

        n = HW * _A
        npad = 1 << int(np.ceil(np.log2(n)))
        spad = jnp.pad(sflat, ((0, 0), (0, npad - n)),
                       constant_values=_NEG).reshape(B, npad // 128, 128)
        ks, vs = _sort_desc(spad)
        k = min(_PRE, n)
        sc_sorted.append(ks.reshape(B, npad)[:, :k])
        idx_sorted.append(vs.reshape(B, npad)[:, :k] + offset)
        offset += n

    s_cat = jnp.concatenate(sc_sorted, axis=1)
    i_cat = jnp.concatenate(idx_sorted, axis=1)
    table = jnp.concatenate(box_tables, axis=1)

    boxes_c = _gather_boxes(table, i_cat)

    ncand = s_cat.shape[1]
    npad = 4096
    spad = jnp.pad(s_cat, ((0, 0), (0, npad - ncand)),
                   constant_values=_NEG).reshape(B, npad // 128, 128)
    ks, vs = _sort_desc(spad)
    s_sorted = ks.reshape(B, npad)[:, :ncand]
    perm = vs.reshape(B, npad)[:, :ncand]
    boxes_s = _gather_boxes(boxes_c, perm)

    brow = jnp.pad(boxes_s, ((0, 0), (0, _NMS_N - ncand), (0, 0)))
    s_in = jnp.pad(s_sorted, ((0, 0), (0, _NMS_N - ncand)),
                   constant_values=_NEG).reshape(B, 1, _NMS_N)
    bcol = jnp.transpose(brow, (0, 2, 1))
    return _nms_topk(brow, bcol, s_in)
